# Initial kernel scaffold; baseline (speedup 1.0000x reference)
import jax
import jax.numpy as jnp
from jax.experimental import pallas as pl


def kernel(item_emb, time_seq, pos_table, time_table, time_scale, ln_gamma, ln_beta):
    raise NotImplementedError("write your pallas kernel here")


# speedup vs baseline: 6.0119x; 6.0119x over previous
import jax
import jax.numpy as jnp
from jax import lax
from jax.experimental import pallas as pl

_B, _S, _H = 1024, 200, 128
_BUCKET = 2048
_SBLK = 8
_HI = _BUCKET // 128


def _fused_kernel(ts_ref, prev_ref, tab_ref, pos_ref, scale_ref, gamma_ref,
                  beta_ref, item_ref, out_ref):
    delta = jnp.maximum(ts_ref[...] - prev_ref[...], 0)
    idx_blk = jnp.bitwise_and(delta, _BUCKET - 1)
    hi = idx_blk >> 7
    lo = idx_blk & 127

    oh_hi = (lax.broadcasted_iota(jnp.int32, (_SBLK, _B, _HI), 2)
             == hi[:, :, None]).astype(jnp.float32)
    oh_lo = (lax.broadcasted_iota(jnp.int32, (_SBLK, _B, 128), 2)
             == lo[:, :, None]).astype(jnp.float32)
    counts = lax.dot_general(oh_hi, oh_lo,
                             (((1,), (1,)), ((0,), (0,))),
                             preferred_element_type=jnp.float32)

    pos_off = jnp.dot(counts.reshape(_SBLK, _BUCKET), tab_ref[...],
                      preferred_element_type=jnp.float32) * (1.0 / _B)
    calib = pos_ref[...] + scale_ref[0, 0] * pos_off

    x = item_ref[...] + calib[None, :, :]
    mu = jnp.mean(x, axis=-1, keepdims=True)
    xc = x - mu
    var = jnp.mean(xc * xc, axis=-1, keepdims=True)
    y = xc * lax.rsqrt(var + 1e-5)
    out_ref[...] = y * gamma_ref[0, :][None, None, :] + beta_ref[0, :][None, None, :]


def kernel(item_emb, time_seq, pos_table, time_table, time_scale, ln_gamma, ln_beta):
    ts_t = time_seq.T
    prev_t = jnp.concatenate([jnp.zeros((1, _B), jnp.int32), ts_t[:-1, :]], axis=0)
    scale = jnp.reshape(time_scale, (1, 1)).astype(jnp.float32)
    gamma = jnp.reshape(ln_gamma, (1, _H))
    beta = jnp.reshape(ln_beta, (1, _H))

    grid = (_S // _SBLK,)
    return pl.pallas_call(
        _fused_kernel,
        grid=grid,
        in_specs=[
            pl.BlockSpec((_SBLK, _B), lambda i: (i, 0)),
            pl.BlockSpec((_SBLK, _B), lambda i: (i, 0)),
            pl.BlockSpec((_BUCKET, _H), lambda i: (0, 0)),
            pl.BlockSpec((_SBLK, _H), lambda i: (i, 0)),
            pl.BlockSpec((1, 1), lambda i: (0, 0)),
            pl.BlockSpec((1, _H), lambda i: (0, 0)),
            pl.BlockSpec((1, _H), lambda i: (0, 0)),
            pl.BlockSpec((_B, _SBLK, _H), lambda i: (0, i, 0)),
        ],
        out_specs=pl.BlockSpec((_B, _SBLK, _H), lambda i: (0, i, 0)),
        out_shape=jax.ShapeDtypeStruct((_B, _S, _H), jnp.float32),
    )(ts_t, prev_t, time_table, pos_table, scale, gamma, beta, item_emb)


# speedup vs baseline: 6.9045x; 1.1485x over previous
import functools

import jax
import jax.numpy as jnp
from jax import lax
from jax.experimental import pallas as pl
from jax.experimental.pallas import tpu as pltpu
from jax.experimental.pallas import tpu_sc as plsc

_B, _S, _H = 1024, 200, 128
_BUCKET = 2048
_SBLK = 8

_NW = 32
_SPT = 8
_SPAD = _NW * _SPT
_CHUNKS = _B // 16


def _sc_hist_kernel(ts_hbm, prev_hbm, out_hbm, ts_v, prev_v, hist_v):
    wid = lax.axis_index("s") * 2 + lax.axis_index("c")
    base = wid * _SPT
    pltpu.sync_copy(ts_hbm.at[pl.ds(base, _SPT)], ts_v)
    pltpu.sync_copy(prev_hbm.at[pl.ds(base, _SPT)], prev_v)

    zeros = jnp.zeros((16,), jnp.float32)

    def _zero(i, _):
        hist_v[pl.ds(i * 16, 16)] = zeros
        return 0

    lax.fori_loop(0, _SPT * _BUCKET // 16, _zero, 0)

    ones = jnp.full((16,), 1.0, jnp.float32)
    for s in range(_SPT):
        def _acc(i, _, s=s):
            t = ts_v[s, pl.ds(i * 16, 16)]
            p = prev_v[s, pl.ds(i * 16, 16)]
            delta = jnp.maximum(t - p, 0)
            idx = jnp.bitwise_and(delta, _BUCKET - 1) + s * _BUCKET
            plsc.addupdate_scatter(hist_v, [idx], ones)
            return 0

        lax.fori_loop(0, _CHUNKS, _acc, 0)

    pltpu.sync_copy(hist_v, out_hbm.at[pl.ds(base * _BUCKET, _SPT * _BUCKET)])


def _sc_histogram(ts_pad, prev_pad):
    mesh = plsc.VectorSubcoreMesh(core_axis_name="c", subcore_axis_name="s")
    f = functools.partial(
        pl.kernel,
        out_type=jax.ShapeDtypeStruct((_SPAD * _BUCKET,), jnp.float32),
        mesh=mesh,
        scratch_types=[
            pltpu.VMEM((_SPT, _B), jnp.int32),
            pltpu.VMEM((_SPT, _B), jnp.int32),
            pltpu.VMEM((_SPT * _BUCKET,), jnp.float32),
        ],
        compiler_params=pltpu.CompilerParams(needs_layout_passes=False),
    )(_sc_hist_kernel)
    return f(ts_pad, prev_pad)


def _tc_fused_kernel(cnt_ref, tab_ref, pos_ref, scale_ref, gamma_ref, beta_ref,
                     item_ref, out_ref):
    pos_off = jnp.dot(cnt_ref[...], tab_ref[...],
                      preferred_element_type=jnp.float32) * (1.0 / _B)
    calib = pos_ref[...] + scale_ref[0, 0] * pos_off

    x = item_ref[...] + calib[None, :, :]
    mu = jnp.mean(x, axis=-1, keepdims=True)
    xc = x - mu
    var = jnp.mean(xc * xc, axis=-1, keepdims=True)
    y = xc * lax.rsqrt(var + 1e-5)
    out_ref[...] = y * gamma_ref[0, :][None, None, :] + beta_ref[0, :][None, None, :]


def kernel(item_emb, time_seq, pos_table, time_table, time_scale, ln_gamma, ln_beta):
    ts_t = time_seq.T
    prev_t = jnp.concatenate([jnp.zeros((1, _B), jnp.int32), ts_t[:-1, :]], axis=0)
    pad = jnp.zeros((_SPAD - _S, _B), jnp.int32)
    counts = _sc_histogram(jnp.concatenate([ts_t, pad], axis=0),
                           jnp.concatenate([prev_t, pad], axis=0))
    counts = counts.reshape(_SPAD, _BUCKET)[:_S]

    scale = jnp.reshape(time_scale, (1, 1)).astype(jnp.float32)
    gamma = jnp.reshape(ln_gamma, (1, _H))
    beta = jnp.reshape(ln_beta, (1, _H))

    grid = (_S // _SBLK,)
    return pl.pallas_call(
        _tc_fused_kernel,
        grid=grid,
        in_specs=[
            pl.BlockSpec((_SBLK, _BUCKET), lambda i: (i, 0)),
            pl.BlockSpec((_BUCKET, _H), lambda i: (0, 0)),
            pl.BlockSpec((_SBLK, _H), lambda i: (i, 0)),
            pl.BlockSpec((1, 1), lambda i: (0, 0)),
            pl.BlockSpec((1, _H), lambda i: (0, 0)),
            pl.BlockSpec((1, _H), lambda i: (0, 0)),
            pl.BlockSpec((_B, _SBLK, _H), lambda i: (0, i, 0)),
        ],
        out_specs=pl.BlockSpec((_B, _SBLK, _H), lambda i: (0, i, 0)),
        out_shape=jax.ShapeDtypeStruct((_B, _S, _H), jnp.float32),
    )(counts, time_table, pos_table, scale, gamma, beta, item_emb)


# speedup vs baseline: 7.0813x; 1.0256x over previous
import functools

import jax
import jax.numpy as jnp
from jax import lax
from jax.experimental import pallas as pl
from jax.experimental.pallas import tpu as pltpu
from jax.experimental.pallas import tpu_sc as plsc

_B, _S, _H = 1024, 200, 128
_BUCKET = 2048
_SBLK = 8

_NW = 32
_SPT = 8
_SPAD = _NW * _SPT
_CHUNKS = _B // 16


def _sc_hist_kernel(ts_hbm, prev_hbm, out_hbm, ts_v, prev_v, hist_v):
    wid = lax.axis_index("s") * 2 + lax.axis_index("c")
    base = wid * _SPT
    pltpu.sync_copy(ts_hbm.at[pl.ds(base, _SPT)], ts_v)
    pltpu.sync_copy(prev_hbm.at[pl.ds(base, _SPT)], prev_v)

    zeros = jnp.zeros((16,), jnp.float32)

    def _zero(i, _):
        for u in range(16):
            hist_v[pl.ds(i * 256 + u * 16, 16)] = zeros
        return 0

    lax.fori_loop(0, _SPT * _BUCKET // 256, _zero, 0)

    ones = jnp.full((16,), 1.0, jnp.float32)
    for s in range(_SPT):
        def _acc(i, _, s=s):
            for u in range(8):
                t = ts_v[s, pl.ds(i * 128 + u * 16, 16)]
                p = prev_v[s, pl.ds(i * 128 + u * 16, 16)]
                delta = jnp.maximum(t - p, 0)
                idx = jnp.bitwise_and(delta, _BUCKET - 1) + s * _BUCKET
                plsc.addupdate_scatter(hist_v, [idx], ones)
            return 0

        lax.fori_loop(0, _CHUNKS // 8, _acc, 0)

    pltpu.sync_copy(hist_v, out_hbm.at[pl.ds(base * _BUCKET, _SPT * _BUCKET)])


def _sc_histogram(ts_pad, prev_pad):
    mesh = plsc.VectorSubcoreMesh(core_axis_name="c", subcore_axis_name="s")
    f = functools.partial(
        pl.kernel,
        out_type=jax.ShapeDtypeStruct((_SPAD * _BUCKET,), jnp.float32),
        mesh=mesh,
        scratch_types=[
            pltpu.VMEM((_SPT, _B), jnp.int32),
            pltpu.VMEM((_SPT, _B), jnp.int32),
            pltpu.VMEM((_SPT * _BUCKET,), jnp.float32),
        ],
        compiler_params=pltpu.CompilerParams(needs_layout_passes=False),
    )(_sc_hist_kernel)
    return f(ts_pad, prev_pad)


def _tc_fused_kernel(cnt_ref, tab_ref, pos_ref, scale_ref, gamma_ref, beta_ref,
                     item_ref, out_ref):
    pos_off = jnp.dot(cnt_ref[...], tab_ref[...],
                      preferred_element_type=jnp.float32) * (1.0 / _B)
    calib = pos_ref[...] + scale_ref[0, 0] * pos_off

    x = item_ref[...] + calib[None, :, :]
    mu = jnp.mean(x, axis=-1, keepdims=True)
    xc = x - mu
    var = jnp.mean(xc * xc, axis=-1, keepdims=True)
    y = xc * lax.rsqrt(var + 1e-5)
    out_ref[...] = y * gamma_ref[0, :][None, None, :] + beta_ref[0, :][None, None, :]


def kernel(item_emb, time_seq, pos_table, time_table, time_scale, ln_gamma, ln_beta):
    ts_t = time_seq.T
    prev_t = jnp.concatenate([jnp.zeros((1, _B), jnp.int32), ts_t[:-1, :]], axis=0)
    pad = jnp.zeros((_SPAD - _S, _B), jnp.int32)
    counts = _sc_histogram(jnp.concatenate([ts_t, pad], axis=0),
                           jnp.concatenate([prev_t, pad], axis=0))
    counts = counts.reshape(_SPAD, _BUCKET)[:_S]

    scale = jnp.reshape(time_scale, (1, 1)).astype(jnp.float32)
    gamma = jnp.reshape(ln_gamma, (1, _H))
    beta = jnp.reshape(ln_beta, (1, _H))

    grid = (_S // _SBLK,)
    return pl.pallas_call(
        _tc_fused_kernel,
        grid=grid,
        in_specs=[
            pl.BlockSpec((_SBLK, _BUCKET), lambda i: (i, 0)),
            pl.BlockSpec((_BUCKET, _H), lambda i: (0, 0)),
            pl.BlockSpec((_SBLK, _H), lambda i: (i, 0)),
            pl.BlockSpec((1, 1), lambda i: (0, 0)),
            pl.BlockSpec((1, _H), lambda i: (0, 0)),
            pl.BlockSpec((1, _H), lambda i: (0, 0)),
            pl.BlockSpec((_B, _SBLK, _H), lambda i: (0, i, 0)),
        ],
        out_specs=pl.BlockSpec((_B, _SBLK, _H), lambda i: (0, i, 0)),
        out_shape=jax.ShapeDtypeStruct((_B, _S, _H), jnp.float32),
    )(counts, time_table, pos_table, scale, gamma, beta, item_emb)


# speedup vs baseline: 7.4707x; 1.0550x over previous
import functools

import jax
import jax.numpy as jnp
from jax import lax
from jax.experimental import pallas as pl
from jax.experimental.pallas import tpu as pltpu
from jax.experimental.pallas import tpu_sc as plsc

_B, _S, _H = 1024, 200, 128
_BUCKET = 2048
_SBLK = 16

_NW = 32
_SPT = 8
_SPAD = _NW * _SPT
_CHUNKS = _B // 16


def _sc_hist_kernel(ts_hbm, prev_hbm, out_hbm, ts_v, prev_v, hist_v):
    wid = lax.axis_index("s") * 2 + lax.axis_index("c")
    base = wid * _SPT
    pltpu.sync_copy(ts_hbm.at[pl.ds(base, _SPT)], ts_v)
    pltpu.sync_copy(prev_hbm.at[pl.ds(base, _SPT)], prev_v)

    zeros = jnp.zeros((16,), jnp.float32)

    def _zero(i, _):
        for u in range(16):
            hist_v[pl.ds(i * 256 + u * 16, 16)] = zeros
        return 0

    lax.fori_loop(0, _SPT * _BUCKET // 256, _zero, 0)

    ones = jnp.full((16,), 1.0, jnp.float32)
    for s in range(_SPT):
        def _acc(i, _, s=s):
            for u in range(8):
                t = ts_v[s, pl.ds(i * 128 + u * 16, 16)]
                p = prev_v[s, pl.ds(i * 128 + u * 16, 16)]
                delta = jnp.maximum(t - p, 0)
                idx = jnp.bitwise_and(delta, _BUCKET - 1) + s * _BUCKET
                plsc.addupdate_scatter(hist_v, [idx], ones)
            return 0

        lax.fori_loop(0, _CHUNKS // 8, _acc, 0)

    pltpu.sync_copy(hist_v, out_hbm.at[pl.ds(base * _BUCKET, _SPT * _BUCKET)])


def _sc_histogram(ts_pad, prev_pad):
    mesh = plsc.VectorSubcoreMesh(core_axis_name="c", subcore_axis_name="s")
    f = functools.partial(
        pl.kernel,
        out_type=jax.ShapeDtypeStruct((_SPAD * _BUCKET,), jnp.float32),
        mesh=mesh,
        scratch_types=[
            pltpu.VMEM((_SPT, _B), jnp.int32),
            pltpu.VMEM((_SPT, _B), jnp.int32),
            pltpu.VMEM((_SPT * _BUCKET,), jnp.float32),
        ],
        compiler_params=pltpu.CompilerParams(needs_layout_passes=False),
    )(_sc_hist_kernel)
    return f(ts_pad, prev_pad)


def _tc_fused_kernel(cnt_ref, tab_ref, pos_ref, scale_ref, gamma_ref, beta_ref,
                     item_ref, out_ref):
    pos_off = jnp.dot(cnt_ref[...], tab_ref[...],
                      preferred_element_type=jnp.float32) * (1.0 / _B)
    calib = pos_ref[...] + scale_ref[0, 0] * pos_off

    x = item_ref[...] + calib[None, :, :]
    mu = jnp.mean(x, axis=-1, keepdims=True)
    xc = x - mu
    var = jnp.mean(xc * xc, axis=-1, keepdims=True)
    y = xc * lax.rsqrt(var + 1e-5)
    out_ref[...] = y * gamma_ref[0, :][None, None, :] + beta_ref[0, :][None, None, :]


def kernel(item_emb, time_seq, pos_table, time_table, time_scale, ln_gamma, ln_beta):
    ts_t = time_seq.T
    prev_t = jnp.concatenate([jnp.zeros((1, _B), jnp.int32), ts_t[:-1, :]], axis=0)
    pad = jnp.zeros((_SPAD - _S, _B), jnp.int32)
    counts = _sc_histogram(jnp.concatenate([ts_t, pad], axis=0),
                           jnp.concatenate([prev_t, pad], axis=0))
    counts = counts.reshape(_SPAD, _BUCKET)[:_S]

    scale = jnp.reshape(time_scale, (1, 1)).astype(jnp.float32)
    gamma = jnp.reshape(ln_gamma, (1, _H))
    beta = jnp.reshape(ln_beta, (1, _H))

    grid = (pl.cdiv(_S, _SBLK),)
    return pl.pallas_call(
        _tc_fused_kernel,
        grid=grid,
        in_specs=[
            pl.BlockSpec((_SBLK, _BUCKET), lambda i: (i, 0)),
            pl.BlockSpec((_BUCKET, _H), lambda i: (0, 0)),
            pl.BlockSpec((_SBLK, _H), lambda i: (i, 0)),
            pl.BlockSpec((1, 1), lambda i: (0, 0)),
            pl.BlockSpec((1, _H), lambda i: (0, 0)),
            pl.BlockSpec((1, _H), lambda i: (0, 0)),
            pl.BlockSpec((_B, _SBLK, _H), lambda i: (0, i, 0)),
        ],
        out_specs=pl.BlockSpec((_B, _SBLK, _H), lambda i: (0, i, 0)),
        out_shape=jax.ShapeDtypeStruct((_B, _S, _H), jnp.float32),
    )(counts, time_table, pos_table, scale, gamma, beta, item_emb)


# speedup vs baseline: 7.6896x; 1.0293x over previous
import functools

import jax
import jax.numpy as jnp
from jax import lax
from jax.experimental import pallas as pl
from jax.experimental.pallas import tpu as pltpu
from jax.experimental.pallas import tpu_sc as plsc

_B, _S, _H = 1024, 200, 128
_BUCKET = 2048
_SBLK = 16

_NW = 32
_SPT = 8
_SPAD = _NW * _SPT
_CHUNKS = _B // 16


def _sc_hist_kernel(ts_hbm, out_hbm, ts_v, hist_v, sem):
    wid = lax.axis_index("s") * 2 + lax.axis_index("c")
    base = wid * _SPT
    cp_ts = pltpu.async_copy(ts_hbm.at[pl.ds(base, 16)], ts_v, sem)

    zeros = jnp.zeros((16,), jnp.float32)

    def _zero(i, _):
        for u in range(16):
            hist_v[pl.ds(i * 256 + u * 16, 16)] = zeros
        return 0

    lax.fori_loop(0, _SPT * _BUCKET // 256, _zero, 0)
    cp_ts.wait()

    ones = jnp.full((16,), 1.0, jnp.float32)
    for s in range(_SPT):
        @plsc.parallel_loop(0, _CHUNKS, 1, unroll=16)
        def _acc(i, s=s):
            t = ts_v[s + 1, pl.ds(i * 16, 16)]
            p = ts_v[s, pl.ds(i * 16, 16)]
            delta = jnp.maximum(t - p, 0)
            idx = jnp.bitwise_and(delta, _BUCKET - 1) + s * _BUCKET
            plsc.addupdate_scatter(hist_v, [idx], ones)

    pltpu.sync_copy(hist_v, out_hbm.at[pl.ds(base * _BUCKET, _SPT * _BUCKET)])


def _sc_histogram(ts_z):
    mesh = plsc.VectorSubcoreMesh(core_axis_name="c", subcore_axis_name="s")
    f = functools.partial(
        pl.kernel,
        out_type=jax.ShapeDtypeStruct((_SPAD * _BUCKET,), jnp.float32),
        mesh=mesh,
        scratch_types=[
            pltpu.VMEM((16, _B), jnp.int32),
            pltpu.VMEM((_SPT * _BUCKET,), jnp.float32),
            pltpu.SemaphoreType.DMA,
        ],
        compiler_params=pltpu.CompilerParams(needs_layout_passes=False),
    )(_sc_hist_kernel)
    return f(ts_z)


def _tc_fused_kernel(cnt_ref, tab_ref, pos_ref, scale_ref, gamma_ref, beta_ref,
                     item_ref, out_ref):
    pos_off = jnp.dot(cnt_ref[...], tab_ref[...],
                      preferred_element_type=jnp.float32) * (1.0 / _B)
    calib = pos_ref[...] + scale_ref[0, 0] * pos_off

    x = item_ref[...] + calib[None, :, :]
    mu = jnp.mean(x, axis=-1, keepdims=True)
    xc = x - mu
    var = jnp.mean(xc * xc, axis=-1, keepdims=True)
    y = xc * lax.rsqrt(var + 1e-5)
    out_ref[...] = y * gamma_ref[0, :][None, None, :] + beta_ref[0, :][None, None, :]


def kernel(item_emb, time_seq, pos_table, time_table, time_scale, ln_gamma, ln_beta):
    ts_t = time_seq.T
    zrow = jnp.zeros((1, _B), jnp.int32)
    pad = jnp.zeros((_SPAD + 16 - 1 - _S, _B), jnp.int32)
    counts = _sc_histogram(jnp.concatenate([zrow, ts_t, pad], axis=0))
    counts = counts.reshape(_SPAD, _BUCKET)[:_S]

    scale = jnp.reshape(time_scale, (1, 1)).astype(jnp.float32)
    gamma = jnp.reshape(ln_gamma, (1, _H))
    beta = jnp.reshape(ln_beta, (1, _H))

    grid = (pl.cdiv(_S, _SBLK),)
    return pl.pallas_call(
        _tc_fused_kernel,
        grid=grid,
        in_specs=[
            pl.BlockSpec((_SBLK, _BUCKET), lambda i: (i, 0)),
            pl.BlockSpec((_BUCKET, _H), lambda i: (0, 0)),
            pl.BlockSpec((_SBLK, _H), lambda i: (i, 0)),
            pl.BlockSpec((1, 1), lambda i: (0, 0)),
            pl.BlockSpec((1, _H), lambda i: (0, 0)),
            pl.BlockSpec((1, _H), lambda i: (0, 0)),
            pl.BlockSpec((_B, _SBLK, _H), lambda i: (0, i, 0)),
        ],
        out_specs=pl.BlockSpec((_B, _SBLK, _H), lambda i: (0, i, 0)),
        out_shape=jax.ShapeDtypeStruct((_B, _S, _H), jnp.float32),
    )(counts, time_table, pos_table, scale, gamma, beta, item_emb)


# speedup vs baseline: 7.7430x; 1.0069x over previous
import functools

import jax
import jax.numpy as jnp
from jax import lax
from jax.experimental import pallas as pl
from jax.experimental.pallas import tpu as pltpu
from jax.experimental.pallas import tpu_sc as plsc

_B, _S, _H = 1024, 200, 128
_BUCKET = 2048
_SBLK = 16

_NW = 32
_SPT = 8
_SPAD = _NW * _SPT
_CHUNKS = _B // 16


def _sc_hist_kernel(ts_hbm, out_hbm, ts_v, hist_v, sem):
    wid = lax.axis_index("s") * 2 + lax.axis_index("c")
    base = wid * _SPT
    cp_ts = pltpu.async_copy(ts_hbm.at[pl.ds(base, 16)], ts_v, sem)

    zeros = jnp.zeros((16,), jnp.float32)

    def _zero(i, _):
        for u in range(16):
            hist_v[pl.ds(i * 256 + u * 16, 16)] = zeros
        return 0

    lax.fori_loop(0, _SPT * _BUCKET // 256, _zero, 0)
    cp_ts.wait()

    ones = jnp.full((16,), 1.0, jnp.float32)
    for s in range(_SPT):
        @plsc.parallel_loop(0, _CHUNKS, 1, unroll=8)
        def _acc(i, s=s):
            t = ts_v[s + 1, pl.ds(i * 16, 16)]
            p = ts_v[s, pl.ds(i * 16, 16)]
            delta = jnp.maximum(t - p, 0)
            idx = jnp.bitwise_and(delta, _BUCKET - 1) + s * _BUCKET
            plsc.addupdate_scatter(hist_v, [idx], ones)

    pltpu.sync_copy(hist_v, out_hbm.at[pl.ds(base * _BUCKET, _SPT * _BUCKET)])


def _sc_histogram(ts_z):
    mesh = plsc.VectorSubcoreMesh(core_axis_name="c", subcore_axis_name="s")
    f = functools.partial(
        pl.kernel,
        out_type=jax.ShapeDtypeStruct((_SPAD * _BUCKET,), jnp.float32),
        mesh=mesh,
        scratch_types=[
            pltpu.VMEM((16, _B), jnp.int32),
            pltpu.VMEM((_SPT * _BUCKET,), jnp.float32),
            pltpu.SemaphoreType.DMA,
        ],
        compiler_params=pltpu.CompilerParams(needs_layout_passes=False),
    )(_sc_hist_kernel)
    return f(ts_z)


def _tc_fused_kernel(cnt_ref, tab_ref, pos_ref, scale_ref, gamma_ref, beta_ref,
                     item_ref, out_ref):
    pos_off = jnp.dot(cnt_ref[...], tab_ref[...],
                      preferred_element_type=jnp.float32) * (1.0 / _B)
    calib = pos_ref[...] + scale_ref[0, 0] * pos_off

    x = item_ref[...] + calib[None, :, :]
    mu = jnp.mean(x, axis=-1, keepdims=True)
    xc = x - mu
    var = jnp.mean(xc * xc, axis=-1, keepdims=True)
    y = xc * lax.rsqrt(var + 1e-5)
    out_ref[...] = y * gamma_ref[0, :][None, None, :] + beta_ref[0, :][None, None, :]


def kernel(item_emb, time_seq, pos_table, time_table, time_scale, ln_gamma, ln_beta):
    ts_t = time_seq.T
    zrow = jnp.zeros((1, _B), jnp.int32)
    pad = jnp.zeros((_SPAD + 16 - 1 - _S, _B), jnp.int32)
    counts = _sc_histogram(jnp.concatenate([zrow, ts_t, pad], axis=0))
    counts = counts.reshape(_SPAD, _BUCKET)[:_S]

    scale = jnp.reshape(time_scale, (1, 1)).astype(jnp.float32)
    gamma = jnp.reshape(ln_gamma, (1, _H))
    beta = jnp.reshape(ln_beta, (1, _H))

    grid = (pl.cdiv(_S, _SBLK),)
    return pl.pallas_call(
        _tc_fused_kernel,
        grid=grid,
        in_specs=[
            pl.BlockSpec((_SBLK, _BUCKET), lambda i: (i, 0)),
            pl.BlockSpec((_BUCKET, _H), lambda i: (0, 0)),
            pl.BlockSpec((_SBLK, _H), lambda i: (i, 0)),
            pl.BlockSpec((1, 1), lambda i: (0, 0)),
            pl.BlockSpec((1, _H), lambda i: (0, 0)),
            pl.BlockSpec((1, _H), lambda i: (0, 0)),
            pl.BlockSpec((_B, _SBLK, _H), lambda i: (0, i, 0)),
        ],
        out_specs=pl.BlockSpec((_B, _SBLK, _H), lambda i: (0, i, 0)),
        out_shape=jax.ShapeDtypeStruct((_B, _S, _H), jnp.float32),
    )(counts, time_table, pos_table, scale, gamma, beta, item_emb)
